# baseline (speedup 1.0000x reference)
import functools

import jax
import jax.numpy as jnp
from jax import lax
from jax.experimental import pallas as pl
from jax.experimental.pallas import tpu as pltpu
from jax.experimental.pallas import tpu_sc as plsc

NNODES = 50000
L = 16
B = NNODES * L
NC, NS = 2, 16
NW = NC * NS
GBLK = 128
NB = B // GBLK


def _make_sc_gather(D, gsub):
  mesh = plsc.VectorSubcoreMesh(core_axis_name="c", subcore_axis_name="s")
  ch = gsub * GBLK
  nch = B // ch
  assert nch * ch == B
  spw = -(-nch // NW)
  assert spw % 2 == 0

  @functools.partial(
      pl.kernel,
      mesh=mesh,
      out_type=jax.ShapeDtypeStruct((B, D), jnp.bfloat16),
      scratch_types=[
          pltpu.VMEM((2, gsub, GBLK), jnp.int32),
          pltpu.VMEM((2, ch, D), jnp.bfloat16),
          pltpu.SemaphoreType.DMA,
          pltpu.SemaphoreType.DMA,
          pltpu.SemaphoreType.DMA,
          pltpu.SemaphoreType.DMA,
      ],
      compiler_params=pltpu.CompilerParams(use_tc_tiling_on_sc=False),
  )
  def gather_k(table_hbm, idx_hbm, out_hbm, idx_v, rows_v, g0, g1, s0, s1):
    wid = lax.axis_index("s") * NC + lax.axis_index("c")
    gsem = (g0, g1)
    ssem = (s0, s1)

    def fire(s, p):
      c = s * NW + wid

      @pl.when(c < nch)
      def _():
        pltpu.sync_copy(idx_hbm.at[pl.ds(c * gsub, gsub)], idx_v.at[p])
        for j in range(gsub):
          pltpu.async_copy(table_hbm.at[idx_v.at[p, j]],
                           rows_v.at[p, pl.ds(j * GBLK, GBLK)], gsem[p])

    def drain_and_store(s, p):
      c = s * NW + wid

      @pl.when(c < nch)
      def _():
        off = c * ch
        pltpu.make_async_copy(out_hbm.at[pl.ds(0, ch)], rows_v.at[p],
                              gsem[p]).wait()
        pltpu.async_copy(rows_v.at[p], out_hbm.at[pl.ds(off, ch)], ssem[p])

    def wait_store(s, p):
      c = s * NW + wid

      @pl.when(c < nch)
      def _():
        pltpu.make_async_copy(rows_v.at[p], out_hbm.at[pl.ds(0, ch)],
                              ssem[p]).wait()

    fire(0, 0)

    def body(s2, carry):
      for p in (0, 1):
        s = s2 * 2 + p
        q = 1 - p
        @pl.when(s >= 1)
        def _():
          wait_store(s - 1, q)

        fire(s + 1, q)
        drain_and_store(s, p)
      return carry

    lax.fori_loop(0, spw // 2, body, 0)
    wait_store(spw - 1, 1)

  return gather_k


_gather = {D: _make_sc_gather(D, 10) for D in (16, 32, 64)}


def _elu(v):
  return jnp.where(v > 0, v, jnp.exp(v) - 1.0)


def _mm_call(g, wt, b, rows):
  k = g.shape[1]
  cout = wt.shape[1]

  def mm_k(g_ref, wt_ref, b_ref, o_ref):
    acc = lax.dot_general(g_ref[...], wt_ref[...], (((1,), (0,)), ((), ())),
                          preferred_element_type=jnp.float32)
    o_ref[...] = _elu(acc + b_ref[...]).astype(jnp.bfloat16)

  return pl.pallas_call(
      mm_k,
      grid=(NNODES // rows,),
      in_specs=[
          pl.BlockSpec((rows, k), lambda i: (i, 0)),
          pl.BlockSpec((k, cout), lambda i: (0, 0)),
          pl.BlockSpec((1, cout), lambda i: (0, 0)),
      ],
      out_specs=pl.BlockSpec((rows, cout), lambda i: (i, 0)),
      out_shape=jax.ShapeDtypeStruct((NNODES, cout), jnp.bfloat16),
  )(g, wt, b)


def _final_call(g3, w3t, b3, wct, bc, rows):
  k = g3.shape[1]
  cmid = w3t.shape[1]
  cout = wct.shape[1]

  def fin_k(g_ref, w3_ref, b3_ref, wc_ref, bc_ref, o_ref):
    h = lax.dot_general(g_ref[...], w3_ref[...], (((1,), (0,)), ((), ())),
                        preferred_element_type=jnp.float32)
    h = _elu(h + b3_ref[...]).astype(jnp.bfloat16)
    o = lax.dot_general(h, wc_ref[...], (((1,), (0,)), ((), ())),
                        preferred_element_type=jnp.float32)
    o_ref[...] = o + bc_ref[...]

  return pl.pallas_call(
      fin_k,
      grid=(NNODES // rows,),
      in_specs=[
          pl.BlockSpec((rows, k), lambda i: (i, 0)),
          pl.BlockSpec((k, cmid), lambda i: (0, 0)),
          pl.BlockSpec((1, cmid), lambda i: (0, 0)),
          pl.BlockSpec((cmid, cout), lambda i: (0, 0)),
          pl.BlockSpec((1, cout), lambda i: (0, 0)),
      ],
      out_specs=pl.BlockSpec((rows, cout), lambda i: (i, 0)),
      out_shape=jax.ShapeDtypeStruct((NNODES, cout), jnp.float32),
  )(g3, w3t, b3, wct, bc)


def kernel(x, indices, W0, b0, W1, b1, W2, b2, W3, b3, Wf1, bf1, Wf2, bf2):
  idx = indices.reshape(NB, GBLK)
  bf = jnp.bfloat16

  h0 = _mm_call(x.astype(bf), W0.T.astype(bf), b0.reshape(1, -1), 2000)
  g1 = _gather[16](h0, idx).reshape(NNODES, 16 * L)
  h1 = _mm_call(g1, W1.T.astype(bf), b1.reshape(1, -1), 2000)
  g2 = _gather[32](h1, idx).reshape(NNODES, 32 * L)
  h2 = _mm_call(g2, W2.T.astype(bf), b2.reshape(1, -1), 2000)
  g3 = _gather[64](h2, idx).reshape(NNODES, 64 * L)

  wc = Wf2 @ Wf1
  bc = Wf2 @ bf1 + bf2
  return _final_call(g3, W3.T.astype(bf), b3.reshape(1, -1),
                     wc.T.astype(bf), bc.reshape(1, -1), 2000)


# speedup vs baseline: 1.0795x; 1.0795x over previous
import functools

import jax
import jax.numpy as jnp
from jax import lax
from jax.experimental import pallas as pl
from jax.experimental.pallas import tpu as pltpu
from jax.experimental.pallas import tpu_sc as plsc

NNODES = 50000
L = 16
B = NNODES * L
NC, NS = 2, 16
NW = NC * NS
GBLK = 128
NB = B // GBLK


def _make_sc_gather(D, gsub):
  mesh = plsc.VectorSubcoreMesh(core_axis_name="c", subcore_axis_name="s")
  ch = gsub * GBLK
  nch = B // ch
  assert nch * ch == B
  spw = -(-nch // NW)
  assert spw % 2 == 0

  @functools.partial(
      pl.kernel,
      mesh=mesh,
      out_type=jax.ShapeDtypeStruct((B, D), jnp.float32),
      scratch_types=[
          pltpu.VMEM((2, gsub, GBLK), jnp.int32),
          pltpu.VMEM((2, ch, D), jnp.float32),
          pltpu.SemaphoreType.DMA,
          pltpu.SemaphoreType.DMA,
          pltpu.SemaphoreType.DMA,
          pltpu.SemaphoreType.DMA,
      ],
      compiler_params=pltpu.CompilerParams(use_tc_tiling_on_sc=False),
  )
  def gather_k(table_hbm, idx_hbm, out_hbm, idx_v, rows_v, g0, g1, s0, s1):
    wid = lax.axis_index("s") * NC + lax.axis_index("c")
    gsem = (g0, g1)
    ssem = (s0, s1)

    def fire(s, p):
      c = s * NW + wid

      @pl.when(c < nch)
      def _():
        pltpu.sync_copy(idx_hbm.at[pl.ds(c * gsub, gsub)], idx_v.at[p])
        for j in range(gsub):
          pltpu.async_copy(table_hbm.at[idx_v.at[p, j]],
                           rows_v.at[p, pl.ds(j * GBLK, GBLK)], gsem[p])

    def drain_and_store(s, p):
      c = s * NW + wid

      @pl.when(c < nch)
      def _():
        off = c * ch
        pltpu.make_async_copy(out_hbm.at[pl.ds(0, ch)], rows_v.at[p],
                              gsem[p]).wait()
        pltpu.async_copy(rows_v.at[p], out_hbm.at[pl.ds(off, ch)], ssem[p])

    def wait_store(s, p):
      c = s * NW + wid

      @pl.when(c < nch)
      def _():
        pltpu.make_async_copy(rows_v.at[p], out_hbm.at[pl.ds(0, ch)],
                              ssem[p]).wait()

    fire(0, 0)

    def body(s2, carry):
      for p in (0, 1):
        s = s2 * 2 + p
        q = 1 - p
        @pl.when(s >= 1)
        def _():
          wait_store(s - 1, q)

        fire(s + 1, q)
        drain_and_store(s, p)
      return carry

    lax.fori_loop(0, spw // 2, body, 0)
    wait_store(spw - 1, 1)

  return gather_k


_gather = {16: _make_sc_gather(16, 10),
           32: _make_sc_gather(32, 10),
           64: _make_sc_gather(64, 5)}


def _elu(v):
  return jnp.where(v > 0, v, jnp.exp(v) - 1.0)


def _fc0_call(x, w0t, b0, rows):
  def k(x_ref, w_ref, b_ref, o_ref):
    acc = lax.dot_general(x_ref[...], w_ref[...], (((1,), (0,)), ((), ())),
                          preferred_element_type=jnp.float32)
    o_ref[...] = _elu(acc + b_ref[...])

  return pl.pallas_call(
      k,
      grid=(NNODES // rows,),
      in_specs=[
          pl.BlockSpec((rows, 128), lambda i: (i, 0)),
          pl.BlockSpec((128, 16), lambda i: (0, 0)),
          pl.BlockSpec((1, 16), lambda i: (0, 0)),
      ],
      out_specs=pl.BlockSpec((rows, 16), lambda i: (i, 0)),
      out_shape=jax.ShapeDtypeStruct((NNODES, 16), jnp.float32),
  )(x, w0t, b0)


def _conv_mm_call(gp, wt, b, rows):
  k = wt.shape[0]
  cout = wt.shape[1]
  p = k // 128
  nb = NNODES // rows

  def mm_k(g_ref, w_ref, b_ref, o_ref, acc_ref):
    cb = pl.program_id(1)

    @pl.when(cb == 0)
    def _():
      acc_ref[...] = jnp.zeros_like(acc_ref)

    acc_ref[...] += lax.dot_general(
        g_ref[...], w_ref[...], (((1,), (0,)), ((), ())),
        preferred_element_type=jnp.float32)

    @pl.when(cb == p - 1)
    def _():
      o_ref[...] = _elu(acc_ref[...] + b_ref[...])

  return pl.pallas_call(
      mm_k,
      grid=(nb, p),
      in_specs=[
          pl.BlockSpec((rows, 128), lambda i, cb: (cb * nb + i, 0)),
          pl.BlockSpec((128, cout), lambda i, cb: (cb, 0)),
          pl.BlockSpec((1, cout), lambda i, cb: (0, 0)),
      ],
      out_specs=pl.BlockSpec((rows, cout), lambda i, cb: (i, 0)),
      out_shape=jax.ShapeDtypeStruct((NNODES, cout), jnp.float32),
      scratch_shapes=[pltpu.VMEM((rows, cout), jnp.float32)],
  )(gp, wt, b)


def _final_call(gp3, w3t, b3, wct, bc, rows):
  k = w3t.shape[0]
  p = k // 128
  cmid = w3t.shape[1]
  cout = wct.shape[1]
  nb = NNODES // rows

  def fin_k(g_ref, w3_ref, b3_ref, wc_ref, bc_ref, o_ref, acc_ref):
    cb = pl.program_id(1)

    @pl.when(cb == 0)
    def _():
      acc_ref[...] = jnp.zeros_like(acc_ref)

    acc_ref[...] += lax.dot_general(
        g_ref[...], w3_ref[...], (((1,), (0,)), ((), ())),
        preferred_element_type=jnp.float32)

    @pl.when(cb == p - 1)
    def _():
      h = _elu(acc_ref[...] + b3_ref[...])
      o = lax.dot_general(h, wc_ref[...], (((1,), (0,)), ((), ())),
                          preferred_element_type=jnp.float32)
      o_ref[...] = o + bc_ref[...]

  return pl.pallas_call(
      fin_k,
      grid=(nb, p),
      in_specs=[
          pl.BlockSpec((rows, 128), lambda i, cb: (cb * nb + i, 0)),
          pl.BlockSpec((128, cmid), lambda i, cb: (cb, 0)),
          pl.BlockSpec((1, cmid), lambda i, cb: (0, 0)),
          pl.BlockSpec((cmid, cout), lambda i, cb: (0, 0)),
          pl.BlockSpec((1, cout), lambda i, cb: (0, 0)),
      ],
      out_specs=pl.BlockSpec((rows, cout), lambda i, cb: (i, 0)),
      out_shape=jax.ShapeDtypeStruct((NNODES, cout), jnp.float32),
      scratch_shapes=[pltpu.VMEM((rows, cmid), jnp.float32)],
  )(gp3, w3t, b3, wct, bc)


def _permute_idx(indices, d):
  p = 128 // d
  ncb = L // p
  i3 = indices.reshape(NNODES, ncb, p)
  return jnp.transpose(i3, (1, 0, 2)).reshape(NB, GBLK)


def kernel(x, indices, W0, b0, W1, b1, W2, b2, W3, b3, Wf1, bf1, Wf2, bf2):
  h0 = _fc0_call(x, W0.T, b0.reshape(1, -1), 2000)

  g1 = _gather[16](h0, _permute_idx(indices, 16)).reshape(2 * NNODES, 128)
  h1 = _conv_mm_call(g1, W1.T, b1.reshape(1, -1), 2000)

  g2 = _gather[32](h1, _permute_idx(indices, 32)).reshape(4 * NNODES, 128)
  h2 = _conv_mm_call(g2, W2.T, b2.reshape(1, -1), 2000)

  g3 = _gather[64](h2, _permute_idx(indices, 64)).reshape(8 * NNODES, 128)

  wc = Wf2 @ Wf1
  bc = Wf2 @ bf1 + bf2
  return _final_call(g3, W3.T, b3.reshape(1, -1),
                     wc.T, bc.reshape(1, -1), 2000)
